# baseline (speedup 1.0000x reference)
import functools

import jax
import jax.numpy as jnp
from jax import lax
from jax.experimental import pallas as pl
from jax.experimental.pallas import tpu as pltpu

N = 4096
F = 4096
D = 64
BN = 256
BIG_I = 2**30
BIG_F = 3.0e38


def _top2_body(x_ref, c_ref, i0_ref, i1_ref):
    x = x_ref[...]
    c = c_ref[...]
    xx = jnp.sum(x * x, axis=1, keepdims=True)
    cc = jnp.sum(c * c, axis=1)
    xc = lax.dot_general(x, c, (((1,), (1,)), ((), ())),
                         preferred_element_type=jnp.float32)
    d2 = xx - 2.0 * xc + cc[None, :]
    iota = lax.broadcasted_iota(jnp.int32, (BN, F), 1)
    i1 = jnp.argmin(d2, axis=1).astype(jnp.int32)
    d2b = jnp.where(iota == i1[:, None], BIG_F, d2)
    i2 = jnp.argmin(d2b, axis=1).astype(jnp.int32)
    i0_ref[...] = i1
    i1_ref[...] = i2


def _top2(x, ctrs):
    n = x.shape[0]
    grid = (n // BN,)
    return pl.pallas_call(
        _top2_body,
        grid=grid,
        in_specs=[
            pl.BlockSpec((BN, D), lambda i: (i, 0)),
            pl.BlockSpec((F, D), lambda i: (0, 0)),
        ],
        out_specs=[
            pl.BlockSpec((BN,), lambda i: (i,)),
            pl.BlockSpec((BN,), lambda i: (i,)),
        ],
        out_shape=[
            jax.ShapeDtypeStruct((n,), jnp.int32),
            jax.ShapeDtypeStruct((n,), jnp.int32),
        ],
    )(x, ctrs)


NW = 32
SPW = N // NW
CH = 4
NCH = SPW // CH


def _sc_combine_body(x_hbm, co_hbm, wflat_hbm, idxp_hbm, y_hbm,
                     idxb0, idxb1, wb0, wb1, cob0, cob1, xbuf, ybuf,
                     sem0, sem1, *, spw, nch):
    wid = lax.axis_index("s") * 2 + lax.axis_index("c")
    base = wid * spw
    pltpu.sync_copy(x_hbm.at[pl.ds(base, spw)], xbuf)
    idxbs, wbs, cobs, sems = (idxb0, idxb1), (wb0, wb1), (cob0, cob1), (sem0, sem1)

    def fire(ci, b):
        pltpu.sync_copy(idxp_hbm.at[wid, ci], idxbs[b])
        pltpu.async_copy(wflat_hbm.at[idxbs[b]], wbs[b], sems[b])
        pltpu.async_copy(co_hbm.at[idxbs[b]], cobs[b], sems[b])

    def drain(b):
        pltpu.make_async_copy(wflat_hbm.at[idxbs[b]], wbs[b], sems[b]).wait()
        pltpu.make_async_copy(co_hbm.at[idxbs[b]], cobs[b], sems[b]).wait()

    def compute(ci, b):
        wb, cob = wbs[b], cobs[b]

        def sbody(s, carry):
            row = ci * CH + s
            acc = (jnp.zeros((16,), jnp.float32),) * 4
            for k in range(2):
                r = k * CH + s

                def dqbody(dq, a):
                    xchunk = (xbuf[row, pl.ds(dq * 16, 16)]
                              - cob[r, pl.ds(dq * 16, 16)])
                    for j in range(16):
                        xv = xchunk[j]
                        a = tuple(
                            a[q] + xv * wb[r, pl.ds(dq * 1024 + j * 64 + q * 16, 16)]
                            for q in range(4))
                    return a

                acc = lax.fori_loop(0, 4, dqbody, acc)
            for q in range(4):
                ybuf[row, pl.ds(q * 16, 16)] = (
                    acc[q] + cob[s, pl.ds(D + q * 16, 16)]
                    + cob[CH + s, pl.ds(D + q * 16, 16)])
            return carry

        lax.fori_loop(0, CH, sbody, 0)

    fire(0, 0)

    def gbody(g, carry):
        ci1 = 2 * g + 1
        fire(ci1, 1)
        drain(0)
        compute(2 * g, 0)

        @pl.when(g < nch // 2 - 1)
        def _():
            fire(ci1 + 1, 0)

        drain(1)
        compute(ci1, 1)
        return carry

    lax.fori_loop(0, nch // 2, gbody, 0)
    pltpu.sync_copy(ybuf, y_hbm.at[pl.ds(base, spw)])


def _sc_combine(x, co, wflat, idxp):
    from jax.experimental.pallas import tpu_sc as plsc
    nh = x.shape[0]
    spw = nh // NW
    nch = spw // CH
    mesh = plsc.VectorSubcoreMesh(
        core_axis_name="c", subcore_axis_name="s", num_cores=2, num_subcores=16)
    return pl.kernel(
        functools.partial(_sc_combine_body, spw=spw, nch=nch),
        out_type=jax.ShapeDtypeStruct((nh, D), jnp.float32),
        mesh=mesh,
        scratch_types=[
            pltpu.VMEM((2 * CH,), jnp.int32),
            pltpu.VMEM((2 * CH,), jnp.int32),
            pltpu.VMEM((2 * CH, D * D), jnp.float32),
            pltpu.VMEM((2 * CH, D * D), jnp.float32),
            pltpu.VMEM((2 * CH, 2 * D), jnp.float32),
            pltpu.VMEM((2 * CH, 2 * D), jnp.float32),
            pltpu.VMEM((spw, D), jnp.float32),
            pltpu.VMEM((spw, D), jnp.float32),
            pltpu.SemaphoreType.DMA,
            pltpu.SemaphoreType.DMA,
        ],
    )(x, co, wflat, idxp)


def kernel(x, ctrs, wts, offsets):
    co = jnp.concatenate([ctrs, offsets], axis=1)
    perm = jnp.arange(F, dtype=jnp.int32) ^ 1
    wflat = jnp.take(wts.reshape(F, D * D), perm, axis=0)
    nh = N // 2
    ys = []
    for h in range(2):
        xh = lax.slice_in_dim(x, h * nh, (h + 1) * nh, axis=0)
        i0, i1 = _top2(xh, ctrs)
        i0 = i0 ^ 1
        i1 = i1 ^ 1
        nch = nh // NW // CH
        idxp = (jnp.stack([i0, i1])
                .reshape(2, NW, nch, CH)
                .transpose(1, 2, 0, 3)
                .reshape(NW, nch, 2 * CH))
        ys.append(_sc_combine(xh, co, wflat, idxp))
    return jnp.concatenate(ys, axis=0)


# speedup vs baseline: 1.2540x; 1.2540x over previous
import functools

import jax
import jax.numpy as jnp
from jax import lax
from jax.experimental import pallas as pl
from jax.experimental.pallas import tpu as pltpu

N = 4096
F = 4096
D = 64
BN = 256
BIG_I = 2**30
BIG_F = 3.0e38


def _top2_body(x_ref, c_ref, i0_ref, i1_ref):
    x = x_ref[...]
    c = c_ref[...]
    xx = jnp.sum(x * x, axis=1, keepdims=True)
    cc = jnp.sum(c * c, axis=1)
    xc = lax.dot_general(x, c, (((1,), (1,)), ((), ())),
                         preferred_element_type=jnp.float32)
    d2 = xx - 2.0 * xc + cc[None, :]
    iota = lax.broadcasted_iota(jnp.int32, (BN, F), 1)
    i1 = jnp.argmin(d2, axis=1).astype(jnp.int32)
    d2b = jnp.where(iota == i1[:, None], BIG_F, d2)
    i2 = jnp.argmin(d2b, axis=1).astype(jnp.int32)
    i0_ref[...] = i1
    i1_ref[...] = i2


def _top2(x, ctrs):
    n = x.shape[0]
    grid = (n // BN,)
    return pl.pallas_call(
        _top2_body,
        grid=grid,
        in_specs=[
            pl.BlockSpec((BN, D), lambda i: (i, 0)),
            pl.BlockSpec((F, D), lambda i: (0, 0)),
        ],
        out_specs=[
            pl.BlockSpec((BN,), lambda i: (i,)),
            pl.BlockSpec((BN,), lambda i: (i,)),
        ],
        out_shape=[
            jax.ShapeDtypeStruct((n,), jnp.int32),
            jax.ShapeDtypeStruct((n,), jnp.int32),
        ],
    )(x, ctrs)


NW = 32
SPW = N // NW
CH = 4
NCH = SPW // CH


def _sc_combine_body(x_hbm, co_hbm, wflat_hbm, idxp_hbm, y_hbm,
                     idxb0, idxb1, wb0, wb1, cob0, cob1, xbuf, ybuf,
                     sem0, sem1, *, spw, nch):
    wid = lax.axis_index("s") * 2 + lax.axis_index("c")
    base = wid * spw
    pltpu.sync_copy(x_hbm.at[pl.ds(base, spw)], xbuf)
    idxbs, wbs, cobs, sems = (idxb0, idxb1), (wb0, wb1), (cob0, cob1), (sem0, sem1)

    def fire(ci, b):
        pltpu.sync_copy(idxp_hbm.at[wid, ci], idxbs[b])
        pltpu.async_copy(wflat_hbm.at[idxbs[b]], wbs[b], sems[b])
        pltpu.async_copy(co_hbm.at[idxbs[b]], cobs[b], sems[b])

    def drain(b):
        pltpu.make_async_copy(wflat_hbm.at[idxbs[b]], wbs[b], sems[b]).wait()
        pltpu.make_async_copy(co_hbm.at[idxbs[b]], cobs[b], sems[b]).wait()

    def compute(ci, b):
        wb, cob = wbs[b], cobs[b]

        def sbody(s, carry):
            row = ci * CH + s
            acc = (jnp.zeros((16,), jnp.float32),) * 4
            for k in range(2):
                r = k * CH + s

                def dqbody(dq, a):
                    xchunk = (xbuf[row, pl.ds(dq * 16, 16)]
                              - cob[r, pl.ds(dq * 16, 16)])
                    for j in range(16):
                        xv = xchunk[j]
                        a = tuple(
                            a[q] + xv * wb[r, pl.ds(dq * 1024 + j * 64 + q * 16, 16)]
                            for q in range(4))
                    return a

                acc = lax.fori_loop(0, 4, dqbody, acc)
            for q in range(4):
                ybuf[row, pl.ds(q * 16, 16)] = (
                    acc[q] + cob[s, pl.ds(D + q * 16, 16)]
                    + cob[CH + s, pl.ds(D + q * 16, 16)])
            return carry

        lax.fori_loop(0, CH, sbody, 0)

    fire(0, 0)

    def gbody(g, carry):
        ci1 = 2 * g + 1
        fire(ci1, 1)
        drain(0)
        compute(2 * g, 0)

        @pl.when(g < nch // 2 - 1)
        def _():
            fire(ci1 + 1, 0)

        drain(1)
        compute(ci1, 1)
        return carry

    lax.fori_loop(0, nch // 2, gbody, 0)
    pltpu.sync_copy(ybuf, y_hbm.at[pl.ds(base, spw)])


def _sc_combine(x, co, wflat, idxp):
    from jax.experimental.pallas import tpu_sc as plsc
    nh = x.shape[0]
    spw = nh // NW
    nch = spw // CH
    mesh = plsc.VectorSubcoreMesh(
        core_axis_name="c", subcore_axis_name="s", num_cores=2, num_subcores=16)
    return pl.kernel(
        functools.partial(_sc_combine_body, spw=spw, nch=nch),
        out_type=jax.ShapeDtypeStruct((nh, D), jnp.float32),
        mesh=mesh,
        scratch_types=[
            pltpu.VMEM((2 * CH,), jnp.int32),
            pltpu.VMEM((2 * CH,), jnp.int32),
            pltpu.VMEM((2 * CH, D * D), jnp.float32),
            pltpu.VMEM((2 * CH, D * D), jnp.float32),
            pltpu.VMEM((2 * CH, 2 * D), jnp.float32),
            pltpu.VMEM((2 * CH, 2 * D), jnp.float32),
            pltpu.VMEM((spw, D), jnp.float32),
            pltpu.VMEM((spw, D), jnp.float32),
            pltpu.SemaphoreType.DMA,
            pltpu.SemaphoreType.DMA,
        ],
    )(x, co, wflat, idxp)


def kernel(x, ctrs, wts, offsets):
    co = jnp.concatenate([ctrs, offsets], axis=1)
    wflat = wts.reshape(F, D * D)
    nh = N // 2
    ys = []
    for h in range(2):
        xh = lax.slice_in_dim(x, h * nh, (h + 1) * nh, axis=0)
        i0, i1 = _top2(xh, ctrs)
        nch = nh // NW // CH
        idxp = (jnp.stack([i0, i1])
                .reshape(2, NW, nch, CH)
                .transpose(1, 2, 0, 3)
                .reshape(NW, nch, 2 * CH))
        ys.append(_sc_combine(xh, co, wflat, idxp))
    return jnp.concatenate(ys, axis=0)
